# Initial kernel scaffold; baseline (speedup 1.0000x reference)
import jax
import jax.numpy as jnp
from jax.experimental import pallas as pl


def kernel(anchors, gt_bboxes):
    raise NotImplementedError("write your pallas kernel here")


# speedup vs baseline: 6.1968x; 6.1968x over previous
import jax
import jax.numpy as jnp
from jax.experimental import pallas as pl
from jax.experimental.pallas import tpu as pltpu

_INF = 100000000.0
_LEVEL_SIZES = (16384, 4096, 1024, 256, 64)
_LEVEL_PAD = (16384, 4096, 1024, 256, 128)
_K = 9
_G = 100
_N = sum(_LEVEL_SIZES)
_NP = sum(_LEVEL_PAD)
_NCAND = _K * len(_LEVEL_SIZES)


def _level_topk_mark(dl):
    n = dl.shape[1]
    lane = jax.lax.broadcasted_iota(jnp.int32, (_G, n), 1).astype(jnp.float32)

    def body(_, d):
        m = jnp.min(d, axis=1, keepdims=True)
        first = jnp.min(jnp.where(d == m, lane, jnp.float32(1e9)),
                        axis=1, keepdims=True)
        return jnp.where(lane == first, jnp.float32(jnp.inf), d)

    return jax.lax.fori_loop(0, _K, body, dl)


def _assign_kernel(at_ref, gt_ref, maxov_ref, argmax_ref, dist_ref):
    at = at_ref[...]
    gt = gt_ref[...]

    ax0 = at[0:1, :]
    ay0 = at[1:2, :]
    ax1 = at[2:3, :]
    ay1 = at[3:4, :]
    gx0 = gt[:, 0:1]
    gy0 = gt[:, 1:2]
    gx1 = gt[:, 2:3]
    gy1 = gt[:, 3:4]

    acx = (ax0 + ax1) * 0.5
    acy = (ay0 + ay1) * 0.5
    gcx = (gx0 + gx1) * 0.5
    gcy = (gy0 + gy1) * 0.5

    dx = acx - gcx
    dy = acy - gcy
    dist_ref[...] = jnp.sqrt(dx * dx + dy * dy)

    start = 0
    for n_l in _LEVEL_PAD:
        dist_ref[:, start:start + n_l] = _level_topk_mark(
            dist_ref[:, start:start + n_l])
        start += n_l
    cand = jnp.isinf(dist_ref[...])
    candf = cand.astype(jnp.float32)

    iw = jnp.maximum(jnp.minimum(ax1, gx1) - jnp.maximum(ax0, gx0), 0.0)
    ih = jnp.maximum(jnp.minimum(ay1, gy1) - jnp.maximum(ay0, gy0), 0.0)
    inter = iw * ih
    area_a = (ax1 - ax0) * (ay1 - ay0)
    area_g = (gx1 - gx0) * (gy1 - gy0)
    ov = inter / jnp.maximum(area_a + area_g - inter, 1e-6)

    mean = jnp.sum(ov * candf, axis=1, keepdims=True) / jnp.float32(_NCAND)
    dev = (ov - mean) * candf
    var = jnp.sum(dev * dev, axis=1, keepdims=True) / jnp.float32(_NCAND - 1)
    thr = mean + jnp.sqrt(var)

    dmin = jnp.minimum(jnp.minimum(acx - gx0, acy - gy0),
                       jnp.minimum(gx1 - acx, gy1 - acy))
    pos = cand & (ov >= thr) & (dmin > 0.01)
    val = jnp.where(pos, ov, -_INF)

    maxv = jnp.max(val, axis=0, keepdims=True)
    row = jax.lax.broadcasted_iota(jnp.int32, (_G, _NP), 0).astype(jnp.float32)
    first_g = jnp.min(jnp.where(val == maxv, row, jnp.float32(1e9)),
                      axis=0, keepdims=True)
    maxov_ref[...] = maxv
    argmax_ref[...] = jnp.where(maxv != -_INF, first_g, 0.0).astype(jnp.int32)


def kernel(anchors, gt_bboxes):
    at = jnp.full((8, _NP), 2e8, jnp.float32)
    at = at.at[:, :_N].set(0.0).at[:4, :_N].set(anchors.T)
    gt = jnp.zeros((_G, 128), jnp.float32).at[:, :4].set(gt_bboxes)
    maxv, ag = pl.pallas_call(
        _assign_kernel,
        out_shape=(
            jax.ShapeDtypeStruct((1, _NP), jnp.float32),
            jax.ShapeDtypeStruct((1, _NP), jnp.int32),
        ),
        scratch_shapes=[pltpu.VMEM((_G, _NP), jnp.float32)],
    )(at, gt)
    return maxv[0, :_N], ag[0, :_N]


# speedup vs baseline: 6.3933x; 1.0317x over previous
import jax
import jax.numpy as jnp
from jax.experimental import pallas as pl
from jax.experimental.pallas import tpu as pltpu

_INF = 100000000.0
_LEVEL_SIZES = (16384, 4096, 1024, 256, 64)
_LEVEL_PAD = (16384, 4096, 1024, 256, 128)
_K = 9
_G = 100
_N = sum(_LEVEL_SIZES)
_NP = sum(_LEVEL_PAD)
_NCAND = _K * len(_LEVEL_SIZES)


def _level_topk_mark(dl):
    n = dl.shape[1]
    lane = jax.lax.broadcasted_iota(jnp.int32, (_G, n), 1).astype(jnp.float32)

    def body(_, d):
        m = jnp.min(d, axis=1, keepdims=True)
        first = jnp.min(jnp.where(d == m, lane, jnp.float32(1e9)),
                        axis=1, keepdims=True)
        return jnp.where(lane == first, jnp.float32(jnp.inf), d)

    return jax.lax.fori_loop(0, _K, body, dl)


def _assign_kernel(at_ref, gt_ref, maxov_ref, argmax_ref, dist_ref):
    at = at_ref[...]
    gt = gt_ref[...]

    ax0 = at[0:1, :]
    ay0 = at[1:2, :]
    ax1 = at[2:3, :]
    ay1 = at[3:4, :]
    gx0 = gt[:, 0:1]
    gy0 = gt[:, 1:2]
    gx1 = gt[:, 2:3]
    gy1 = gt[:, 3:4]

    acx = (ax0 + ax1) * 0.5
    acy = (ay0 + ay1) * 0.5
    gcx = (gx0 + gx1) * 0.5
    gcy = (gy0 + gy1) * 0.5

    dx = acx - gcx
    dy = acy - gcy
    dist_ref[...] = dx * dx + dy * dy

    start = 0
    for n_l in _LEVEL_PAD:
        dist_ref[:, start:start + n_l] = _level_topk_mark(
            dist_ref[:, start:start + n_l])
        start += n_l
    cand = jnp.isinf(dist_ref[...])
    candf = cand.astype(jnp.float32)

    iw = jnp.maximum(jnp.minimum(ax1, gx1) - jnp.maximum(ax0, gx0), 0.0)
    ih = jnp.maximum(jnp.minimum(ay1, gy1) - jnp.maximum(ay0, gy0), 0.0)
    inter = iw * ih
    area_a = (ax1 - ax0) * (ay1 - ay0)
    area_g = (gx1 - gx0) * (gy1 - gy0)
    ov = inter / jnp.maximum(area_a + area_g - inter, 1e-6)

    mean = jnp.sum(ov * candf, axis=1, keepdims=True) / jnp.float32(_NCAND)
    dev = (ov - mean) * candf
    var = jnp.sum(dev * dev, axis=1, keepdims=True) / jnp.float32(_NCAND - 1)
    thr = mean + jnp.sqrt(var)

    dmin = jnp.minimum(jnp.minimum(acx - gx0, acy - gy0),
                       jnp.minimum(gx1 - acx, gy1 - acy))
    pos = cand & (ov >= thr) & (dmin > 0.01)
    val = jnp.where(pos, ov, -_INF)

    maxv = jnp.max(val, axis=0, keepdims=True)
    row = jax.lax.broadcasted_iota(jnp.int32, (_G, _NP), 0).astype(jnp.float32)
    first_g = jnp.min(jnp.where(val == maxv, row, jnp.float32(1e9)),
                      axis=0, keepdims=True)
    maxov_ref[...] = maxv
    argmax_ref[...] = jnp.where(maxv != -_INF, first_g, 0.0).astype(jnp.int32)


def kernel(anchors, gt_bboxes):
    at = jnp.full((8, _NP), 2e8, jnp.float32)
    at = at.at[:, :_N].set(0.0).at[:4, :_N].set(anchors.T)
    gt = jnp.zeros((_G, 128), jnp.float32).at[:, :4].set(gt_bboxes)
    maxv, ag = pl.pallas_call(
        _assign_kernel,
        out_shape=(
            jax.ShapeDtypeStruct((1, _NP), jnp.float32),
            jax.ShapeDtypeStruct((1, _NP), jnp.int32),
        ),
        scratch_shapes=[pltpu.VMEM((_G, _NP), jnp.float32)],
    )(at, gt)
    return maxv[0, :_N], ag[0, :_N]
